# baseline (speedup 1.0000x reference)
import functools

import jax
import jax.numpy as jnp
from jax import lax
from jax.experimental import pallas as pl
from jax.experimental.pallas import tpu as pltpu
from jax.experimental.pallas import tpu_sc as plsc

N = 10000
E = 320000
D = 128
A = 8

NC = 2
NS = 16
NW = NC * NS
EPW = E // NW
CH = 80
NCHUNK = EPW // CH
NP = 10240
RPT = NP // NS
SR = 32

_BN_SCALE = (1.0 + 1e-5) ** -0.5

_sc_mesh = plsc.VectorSubcoreMesh(core_axis_name="c", subcore_axis_name="s")


@functools.partial(
    pl.kernel,
    out_type=jax.ShapeDtypeStruct((NW, NP), jnp.float32),
    mesh=_sc_mesh,
    scratch_types=[
        pltpu.VMEM((NP,), jnp.float32),
        pltpu.VMEM((EPW,), jnp.int32),
    ],
    compiler_params=pltpu.CompilerParams(needs_layout_passes=False),
)
def _sc_hist(row_hbm, out_hbm, hist, idxs):
    cid = lax.axis_index("c")
    sid = lax.axis_index("s")
    wid = sid * NC + cid
    zero16 = jnp.zeros((16,), jnp.float32)

    def zbody(k, carry):
        hist[pl.ds(k * 16, 16)] = zero16
        return carry

    lax.fori_loop(0, NP // 16, zbody, 0)
    pltpu.sync_copy(row_hbm.at[pl.ds(wid * EPW, EPW)], idxs)
    ones16 = jnp.ones((16,), jnp.float32)

    def body(k, carry):
        idxv = idxs[pl.ds(k * 16, 16)]
        plsc.addupdate_scatter(hist, [idxv], ones16)
        return carry

    lax.fori_loop(0, EPW // 16, body, 0)
    pltpu.sync_copy(hist, out_hbm.at[wid])


@functools.partial(
    pl.kernel,
    out_type=[jax.ShapeDtypeStruct((NP, D), jnp.float32),
              jax.ShapeDtypeStruct((NP, D), jnp.float32)],
    mesh=_sc_mesh,
    scratch_types=[
        pltpu.VMEM_SHARED((NP, D), jnp.float32),
        pltpu.VMEM((SR, D), jnp.float32),
        pltpu.VMEM((EPW,), jnp.int32),
        pltpu.VMEM((NCHUNK, CH), jnp.int32),
        pltpu.VMEM((CH, D), jnp.float32),
        pltpu.VMEM((CH, D), jnp.float32),
        pltpu.SemaphoreType.DMA,
        pltpu.SemaphoreType.DMA,
    ],
    compiler_params=pltpu.CompilerParams(use_tc_tiling_on_sc=False),
)
def _sc_agg(z_hbm, gidx_hbm, col_hbm, zeros_hbm, out0_hbm, out1_hbm,
            acc, stage, idxall, colall, rows0, rows1, sem0, sem1):
    cid = lax.axis_index("c")
    sid = lax.axis_index("s")
    wid = sid * NC + cid
    base = wid * EPW
    pltpu.sync_copy(zeros_hbm, stage)

    def zbody(t, carry):
        pltpu.sync_copy(stage, acc.at[pl.ds(sid * RPT + t * SR, SR)])
        return carry

    lax.fori_loop(0, RPT // SR, zbody, 0)
    pltpu.sync_copy(gidx_hbm.at[pl.ds(base, EPW)], idxall)
    pltpu.sync_copy(col_hbm.at[wid], colall)
    plsc.subcore_barrier()

    def gather(j, rows, sem):
        return pltpu.async_copy(z_hbm.at[idxall.at[pl.ds(j * CH, CH)]], rows, sem)

    def gwait(j, rows, sem):
        pltpu.make_async_copy(z_hbm.at[idxall.at[pl.ds(j * CH, CH)]], rows, sem).wait()

    def scatter(j, rows):
        pltpu.sync_copy(rows, acc.at[colall.at[j]], add=True)

    gather(0, rows0, sem0)

    def body(g, carry):
        j = 2 * g
        gather(j + 1, rows1, sem1)
        gwait(j, rows0, sem0)
        scatter(j, rows0)
        gather(j + 2, rows0, sem0)
        gwait(j + 1, rows1, sem1)
        scatter(j + 1, rows1)
        return carry

    lax.fori_loop(0, (NCHUNK - 1) // 2, body, 0)
    gwait(NCHUNK - 1, rows0, sem0)
    scatter(NCHUNK - 1, rows0)
    plsc.subcore_barrier()

    def wbody(t, carry):
        pltpu.sync_copy(acc.at[pl.ds(sid * RPT + t * SR, SR)], stage)

        @pl.when(cid == 0)
        def _():
            pltpu.sync_copy(stage, out0_hbm.at[pl.ds(sid * RPT + t * SR, SR)])

        @pl.when(cid == 1)
        def _():
            pltpu.sync_copy(stage, out1_hbm.at[pl.ds(sid * RPT + t * SR, SR)])

        return carry

    lax.fori_loop(0, RPT // SR, wbody, 0)


_BR = 1000
_BR2 = 1000


def _deg_dis_inv(hist_blk):
    deg = jnp.sum(hist_blk, axis=1) + 1.0
    return deg, lax.rsqrt(deg), 1.0 / deg


def _table_and_self(x, root, dis, inv, br, z_ref, self_ref):
    self_ref[...] = jnp.maximum(x + root, 0.0) * inv[:, None]
    aval = lax.broadcasted_iota(jnp.int32, (br, A, D), 1).astype(jnp.float32)
    z = jnp.maximum(x[:, None, :] + aval, 0.0) * dis[:, None, None]
    z_ref[...] = z.reshape(br * A, D)


def _k_pre_body(h_ref, w_ref, b_ref, root_ref, hist_ref, z_ref, self_ref):
    _, dis, inv = _deg_dis_inv(hist_ref[...])
    x = lax.dot_general(h_ref[...], w_ref[...], (((1,), (1,)), ((), ())),
                        preferred_element_type=jnp.float32) + b_ref[...]
    _table_and_self(x, root_ref[...], dis, inv, _BR, z_ref, self_ref)


def _k_pre(h, w, b, root, hist):
    return pl.pallas_call(
        _k_pre_body,
        grid=(N // _BR,),
        in_specs=[
            pl.BlockSpec((_BR, D), lambda i: (i, 0)),
            pl.BlockSpec((D, D), lambda i: (0, 0)),
            pl.BlockSpec((1, D), lambda i: (0, 0)),
            pl.BlockSpec((1, D), lambda i: (0, 0)),
            pl.BlockSpec((_BR, NW), lambda i: (i, 0)),
        ],
        out_specs=[
            pl.BlockSpec((_BR * A, D), lambda i: (i, 0)),
            pl.BlockSpec((_BR, D), lambda i: (i, 0)),
        ],
        out_shape=[
            jax.ShapeDtypeStruct((N * A, D), jnp.float32),
            jax.ShapeDtypeStruct((N, D), jnp.float32),
        ],
    )(h, w, b, root, hist)


def _combine(p0, p1, dis, selft, g, be):
    y = (p0 + p1) * dis[:, None] + selft
    return y * (g * _BN_SCALE) + be


def _k_mid_body(p0_ref, p1_ref, self0_ref, g_ref, be_ref, hist_ref,
                w_ref, b_ref, root_ref, z_ref, self_ref):
    _, dis, inv = _deg_dis_inv(hist_ref[...])
    h1 = jnp.maximum(
        _combine(p0_ref[...], p1_ref[...], dis, self0_ref[...],
                 g_ref[...], be_ref[...]), 0.0)
    x = lax.dot_general(h1, w_ref[...], (((1,), (1,)), ((), ())),
                        preferred_element_type=jnp.float32) + b_ref[...]
    _table_and_self(x, root_ref[...], dis, inv, _BR2, z_ref, self_ref)


def _k_mid(pa, pb, self0, g, be, hist, w, b, root):
    return pl.pallas_call(
        _k_mid_body,
        grid=(N // _BR2,),
        in_specs=[
            pl.BlockSpec((_BR2, D), lambda i: (i, 0)),
            pl.BlockSpec((_BR2, D), lambda i: (i, 0)),
            pl.BlockSpec((_BR2, D), lambda i: (i, 0)),
            pl.BlockSpec((1, D), lambda i: (0, 0)),
            pl.BlockSpec((1, D), lambda i: (0, 0)),
            pl.BlockSpec((_BR2, NW), lambda i: (i, 0)),
            pl.BlockSpec((D, D), lambda i: (0, 0)),
            pl.BlockSpec((1, D), lambda i: (0, 0)),
            pl.BlockSpec((1, D), lambda i: (0, 0)),
        ],
        out_specs=[
            pl.BlockSpec((_BR2 * A, D), lambda i: (i, 0)),
            pl.BlockSpec((_BR2, D), lambda i: (i, 0)),
        ],
        out_shape=[
            jax.ShapeDtypeStruct((N * A, D), jnp.float32),
            jax.ShapeDtypeStruct((N, D), jnp.float32),
        ],
    )(pa, pb, self0, g, be, hist, w, b, root)


def _k_post_body(p0_ref, p1_ref, self1_ref, g_ref, be_ref, hist_ref, o_ref):
    _, dis, _ = _deg_dis_inv(hist_ref[...])
    o_ref[...] = _combine(p0_ref[...], p1_ref[...], dis, self1_ref[...],
                          g_ref[...], be_ref[...])


def _k_post(pa, pb, self1, g, be, hist):
    return pl.pallas_call(
        _k_post_body,
        grid=(N // _BR2,),
        in_specs=[
            pl.BlockSpec((_BR2, D), lambda i: (i, 0)),
            pl.BlockSpec((_BR2, D), lambda i: (i, 0)),
            pl.BlockSpec((_BR2, D), lambda i: (i, 0)),
            pl.BlockSpec((1, D), lambda i: (0, 0)),
            pl.BlockSpec((1, D), lambda i: (0, 0)),
            pl.BlockSpec((_BR2, NW), lambda i: (i, 0)),
        ],
        out_specs=pl.BlockSpec((_BR2, D), lambda i: (i, 0)),
        out_shape=jax.ShapeDtypeStruct((N, D), jnp.float32),
    )(pa, pb, self1, g, be, hist)


def kernel(x, edge_index, edge_attr, W0, b0, root0, g0, be0,
           W1, b1, root1, g1, be1):
    row = edge_index[0]
    col = edge_index[1]
    gidx = row * A + edge_attr[:, 0]

    zerosD = jnp.zeros((SR, D), jnp.float32)
    col3 = col.reshape(NW, NCHUNK, CH)

    hist = _sc_hist(row).T

    b0r = b0.reshape(1, D)
    b1r = b1.reshape(1, D)

    z0, self0 = _k_pre(x, W0, b0r, root0, hist)
    p0a, p0b = _sc_agg(z0, gidx, col3, zerosD)
    z1, self1 = _k_mid(p0a, p0b, self0, g0.reshape(1, D), be0.reshape(1, D),
                       hist, W1, b1r, root1)
    p1a, p1b = _sc_agg(z1, gidx, col3, zerosD)
    return _k_post(p1a, p1b, self1, g1.reshape(1, D), be1.reshape(1, D), hist)
